# Initial kernel scaffold; baseline (speedup 1.0000x reference)
import jax
import jax.numpy as jnp
from jax.experimental import pallas as pl


def kernel(init_users_embeddings, init_items_embeddings, adj_indices, adj_values):
    raise NotImplementedError("write your pallas kernel here")


# speedup vs baseline: 2.5585x; 2.5585x over previous
import functools

import jax
import jax.numpy as jnp
from jax import lax
from jax.experimental import pallas as pl
from jax.experimental.pallas import tpu as pltpu
from jax.experimental.pallas import tpu_sc as plsc

N_USERS = 25000
M_ITEMS = 25000
N = N_USERS + M_ITEMS
EMB = 64
NNZ = 800000
LAYERS = 3

NC = 2
NS = 16
HALF = 25000
PAD_HALF = 25088
TPAD = 2 * PAD_HALF
ACC_ROWS = 25216
TRASH = 25088
WB = PAD_HALF // NS
ZCH = ACC_ROWS // NS

NNZ_PAD = 819200
E_TILE = NNZ_PAD // NS
BLK = 256
NBLK = E_TILE // BLK
GSUB = 128
NSUB = BLK // GSUB


def _propagate_layer(table, src, dst, val):
    mesh = plsc.VectorSubcoreMesh(core_axis_name="c", subcore_axis_name="s")

    @functools.partial(
        pl.kernel,
        out_type=jax.ShapeDtypeStruct((TPAD, EMB), jnp.float32),
        mesh=mesh,
        compiler_params=pltpu.CompilerParams(use_tc_tiling_on_sc=False),
        scratch_types=[
            pltpu.VMEM_SHARED((ACC_ROWS, EMB), jnp.float32),
            pltpu.VMEM((BLK,), jnp.int32),
            pltpu.VMEM((BLK,), jnp.int32),
            pltpu.VMEM((BLK,), jnp.float32),
            pltpu.VMEM((BLK,), jnp.int32),
            pltpu.VMEM((NSUB, GSUB), jnp.int32),
            pltpu.VMEM((BLK, EMB), jnp.float32),
            pltpu.SemaphoreType.DMA,
            pltpu.SemaphoreType.DMA,
        ],
    )
    def layer(table_hbm, src_hbm, dst_hbm, val_hbm, out_hbm,
              accum, esrc, edst, eval_, srcx, idxl, rows, gsem, ssem):
        c = lax.axis_index("c")
        s = lax.axis_index("s")
        dst_base = c * HALF

        def zero_rows(r, _):
            for k in range(EMB // 16):
                rows[r, pl.ds(k * 16, 16)] = jnp.zeros((16,), jnp.float32)
            return _
        lax.fori_loop(0, BLK, zero_rows, None)

        def zero_acc(z, _):
            pltpu.sync_copy(rows, accum.at[pl.ds(s * ZCH + z * BLK, BLK)])
            return _
        lax.fori_loop(0, ZCH // BLK, zero_acc, None)
        pltpu.sync_copy(rows.at[pl.ds(0, ZCH % BLK)],
                        accum.at[pl.ds(s * ZCH + (ZCH // BLK) * BLK, ZCH % BLK)])
        plsc.subcore_barrier()

        def block_body(b, _):
            eb = s * E_TILE + b * BLK
            pltpu.sync_copy(src_hbm.at[pl.ds(eb, BLK)], esrc)
            pltpu.sync_copy(dst_hbm.at[pl.ds(eb, BLK)], edst)
            pltpu.sync_copy(val_hbm.at[pl.ds(eb, BLK)], eval_)

            def prep(g, _):
                sv = esrc[pl.ds(g * 16, 16)]
                srcx[pl.ds(g * 16, 16)] = jnp.where(sv >= HALF, sv + (PAD_HALF - HALF), sv)
                dv = edst[pl.ds(g * 16, 16)]
                own = (dv >= dst_base) & (dv < dst_base + HALF)
                lidx = jnp.where(own, dv - dst_base, TRASH)
                idxl[g // 8, pl.ds((g % 8) * 16, 16)] = lidx
                return _
            lax.fori_loop(0, BLK // 16, prep, None)

            gathers = [
                pltpu.async_copy(
                    table_hbm.at[srcx.at[pl.ds(j * GSUB, GSUB)]],
                    rows.at[pl.ds(j * GSUB, GSUB)], gsem)
                for j in range(NSUB)
            ]
            for h in gathers:
                h.wait()

            def scale(g, _):
                vv = eval_[pl.ds(g * 16, 16)]
                for i in range(16):
                    sp = jnp.broadcast_to(vv[i], (16,))
                    r = g * 16 + i
                    for k in range(EMB // 16):
                        rows[r, pl.ds(k * 16, 16)] = rows[r, pl.ds(k * 16, 16)] * sp
                return _
            lax.fori_loop(0, BLK // 16, scale, None)

            scatters = [
                pltpu.async_copy(
                    rows.at[pl.ds(j * GSUB, GSUB)],
                    accum.at[idxl.at[j]], ssem, add=True)
                for j in range(NSUB)
            ]
            for h in scatters:
                h.wait()
            return _

        lax.fori_loop(0, NBLK, block_body, None)
        plsc.subcore_barrier()

        pltpu.sync_copy(accum.at[pl.ds(s * WB, WB)],
                        out_hbm.at[pl.ds(c * PAD_HALF + s * WB, WB)])

    return layer(table, src, dst, val)


def _mean4(t0, t1, t2, t3):
    grid = 14
    rows = TPAD // grid

    def body(a, b, c, d, o):
        o[...] = 0.25 * (a[...] + b[...] + c[...] + d[...])

    spec = pl.BlockSpec((rows, EMB), lambda i: (i, 0))
    return pl.pallas_call(
        body,
        grid=(grid,),
        in_specs=[spec] * 4,
        out_specs=spec,
        out_shape=jax.ShapeDtypeStruct((TPAD, EMB), jnp.float32),
    )(t0, t1, t2, t3)


def kernel(init_users_embeddings, init_items_embeddings, adj_indices, adj_values):
    zpad = jnp.zeros((PAD_HALF - N_USERS, EMB), jnp.float32)
    t0 = jnp.concatenate(
        [init_users_embeddings.astype(jnp.float32), zpad,
         init_items_embeddings.astype(jnp.float32), zpad], axis=0)

    src = adj_indices[0].astype(jnp.int32)
    dst = adj_indices[1].astype(jnp.int32)
    val = adj_values.astype(jnp.float32)
    epad = NNZ_PAD - src.shape[0]
    src = jnp.concatenate([src, jnp.zeros((epad,), jnp.int32)])
    dst = jnp.concatenate([dst, jnp.zeros((epad,), jnp.int32)])
    val = jnp.concatenate([val, jnp.zeros((epad,), jnp.float32)])

    t1 = _propagate_layer(t0, src, dst, val)
    t2 = _propagate_layer(t1, src, dst, val)
    t3 = _propagate_layer(t2, src, dst, val)
    mean = _mean4(t0, t1, t2, t3)

    users_final = mean[:N_USERS]
    items_final = mean[PAD_HALF:PAD_HALF + M_ITEMS]
    return users_final, items_final
